# baseline (speedup 1.0000x reference)
import jax
import jax.numpy as jnp
from jax import lax
from jax.experimental import pallas as pl
from jax.experimental.pallas import tpu as pltpu
from jax.experimental.pallas import tpu_sc as plsc

_NUM_KEYS = 26
_BATCH = 4096
_MAX_LEN = 200
_PW_FLAT = _NUM_KEYS * _MAX_LEN

_NL = 16
_C = 32768
_CG = _C // _NL
_W = 32
_WB = _W * _NL
_NW = 32
_SAMP = 256
_INT_MAX = jnp.iinfo(jnp.int32).max


def _sc_expand(pwf, qq, ss, qs, nchunks, total):
    nfull = nchunks - 1
    tailc = total - nfull * _C
    owner = nfull % _NW
    nsv = qs.shape[0] // _NL
    mesh = plsc.VectorSubcoreMesh(core_axis_name="c", subcore_axis_name="s")

    def body(pw_hbm, qq_hbm, ss_hbm, qs_hbm, out_hbm,
             pw_v, qs_v, q_v, s_v, e_v, sg_v, pg_v, out_v):
        wid = lax.axis_index("s") * 2 + lax.axis_index("c")
        pltpu.sync_copy(pw_hbm, pw_v)
        pltpu.sync_copy(qs_hbm, qs_v)
        zero16 = jnp.zeros((_NL,), jnp.int32)
        iota = lax.iota(jnp.int32, _NL)

        @plsc.parallel_loop(0, _CG, unroll=8)
        def _(k):
            e_v[pl.ds(k * _NL, _NL)] = zero16

        @plsc.parallel_loop(0, _CG // _NL, unroll=8)
        def _(k):
            sg_v[pl.ds(k * _NL, _NL)] = zero16

        def stage(off):
            pltpu.sync_copy(qq_hbm.at[pl.ds(off, _WB)], q_v.at[pl.ds(0, _WB)])
            pltpu.sync_copy(ss_hbm.at[pl.ds(off, _WB)], s_v)

        def scatter_groups(c0):
            for g in range(_W):
                q = q_v[pl.ds(g * _NL, _NL)]
                s = s_v[pl.ds(g * _NL, _NL)]
                m = (q >= c0) & (q < c0 + _C)
                ql = q - c0
                plsc.addupdate_scatter(e_v, [ql], s, mask=m)
                plsc.addupdate_scatter(sg_v, [ql >> 4], s, mask=m)

        def process_chunk(c, c0, dma_words):
            def samp_body(j, acc):
                return acc + jnp.where(qs_v[pl.ds(j * _NL, _NL)] < c0, 1, 0)

            s1 = plsc.cumsum(lax.fori_loop(0, nsv, samp_body, zero16))[_NL - 1]
            base = jnp.maximum(s1 - 1, 0) * _SAMP

            stage(base)

            def cnt_body(j, acc):
                return acc + jnp.where(q_v[pl.ds(j * _NL, _NL)] < c0, 1, 0)

            cnt = plsc.cumsum(lax.fori_loop(0, _W, cnt_body, zero16))[_NL - 1]
            blo = base + cnt
            pprev = q_v[pl.ds(jnp.maximum(cnt - 1, 0), _NL)][0]
            carry = jnp.where(
                blo > 0, ((blo - 1) // _BATCH) * _MAX_LEN - pprev, 0)

            scatter_groups(c0)
            lq0 = q_v[pl.ds(_WB - _NL, _NL)][_NL - 1]

            def wcond(st):
                return st[1] < c0 + _C

            def wbody(st):
                r = st[0]
                stage(base + r * _WB)
                scatter_groups(c0)
                return (r + 1, q_v[pl.ds(_WB - _NL, _NL)][_NL - 1])

            lax.while_loop(wcond, wbody, (jnp.int32(1), lq0))

            def scan_body(k, car):
                v = sg_v[pl.ds(k * _NL, _NL)]
                sg_v[pl.ds(k * _NL, _NL)] = zero16
                inc = plsc.cumsum(v)
                pg_v[pl.ds(k * _NL, _NL)] = inc - v + car
                return car + inc[_NL - 1]

            lax.fori_loop(0, _CG // _NL, scan_body, carry)

            @plsc.parallel_loop(0, _CG, unroll=8)
            def _(k):
                v = e_v[pl.ds(k * _NL, _NL)]
                e_v[pl.ds(k * _NL, _NL)] = zero16
                pk = pg_v[pl.ds(k, _NL)][0]
                flat = plsc.cumsum(v) + (pk + c0 + k * _NL) + iota
                flat = jnp.clip(flat, 0, _PW_FLAT - 1)
                out_v[pl.ds(k * _NL, _NL)] = plsc.load_gather(pw_v, [flat])

            pltpu.sync_copy(out_v.at[pl.ds(0, dma_words)],
                            out_hbm.at[pl.ds(c0, dma_words)])

        def chunk_body(t, _):
            c = wid + t * _NW
            process_chunk(c, c * _C, _C)
            return 0

        my_n = jnp.maximum(0, (nfull - wid + _NW - 1) // _NW)
        lax.fori_loop(0, my_n, chunk_body, 0)

        @pl.when(wid == owner)
        def _():
            process_chunk(nfull, nfull * _C, tailc)

    call = pl.kernel(
        body,
        out_type=jax.ShapeDtypeStruct((total,), jnp.float32),
        mesh=mesh,
        compiler_params=pltpu.CompilerParams(needs_layout_passes=False),
        scratch_types=[
            pltpu.VMEM((_PW_FLAT,), jnp.float32),
            pltpu.VMEM((qs.shape[0],), jnp.int32),
            pltpu.VMEM((_WB + _NL,), jnp.int32),
            pltpu.VMEM((_WB,), jnp.int32),
            pltpu.VMEM((_C,), jnp.int32),
            pltpu.VMEM((_CG,), jnp.int32),
            pltpu.VMEM((_CG + _NL,), jnp.int32),
            pltpu.VMEM((_C,), jnp.float32),
        ],
    )
    return call(pwf, qq, ss, qs)


def kernel(values, lengths, position_weights):
    total = values.shape[0]
    if total == 0:
        return jnp.zeros((0,), jnp.float32)
    n = lengths.shape[0]
    cl = lengths.astype(jnp.int32)
    offs = jnp.concatenate(
        [jnp.zeros((1,), jnp.int32), jnp.cumsum(cl, dtype=jnp.int32)])
    p = offs[:n]
    keyid = jnp.arange(n, dtype=jnp.int32) // _BATCH
    d = keyid * _MAX_LEN - p
    sprime = jnp.concatenate([d[:1], d[1:] - d[:-1]])
    nchunks = -(-total // _C)
    qq = jnp.concatenate([p, jnp.full((_WB,), _INT_MAX, jnp.int32)])
    ss = jnp.concatenate([sprime, jnp.zeros((_WB,), jnp.int32)])
    qs = jnp.concatenate(
        [p[::_SAMP], jnp.full((_NL,), _INT_MAX, jnp.int32)])
    nsp = ((qs.shape[0] + _NL - 1) // _NL) * _NL
    qs = jnp.pad(qs, (0, nsp - qs.shape[0]), constant_values=_INT_MAX)
    pwf = position_weights.reshape(-1).astype(jnp.float32)
    return _sc_expand(pwf, qq, ss, qs, nchunks, total)
